# Initial kernel scaffold; baseline (speedup 1.0000x reference)
import jax
import jax.numpy as jnp
from jax.experimental import pallas as pl


def kernel(x, edge_index, W1, b1, gw1, gb1, gm1, W2, b2, gw2, gb2, gm2, W3, b3, gw3, gb3, gm3):
    raise NotImplementedError("write your pallas kernel here")


# speedup vs baseline: 8.4554x; 8.4554x over previous
import functools

import jax
import jax.numpy as jnp
from jax import lax
from jax.experimental import pallas as pl
from jax.experimental.pallas import tpu as pltpu
from jax.experimental.pallas import tpu_sc as plsc

N = 10000
E = 320000
NPAD = 10240
C = 128
NW = 32
EPW = E // NW
B = 80
NB = EPW // B
ROWS_PER_TILE = NPAD // 16
SNIB = ROWS_PER_TILE // B
DEG_PER_TILE = NPAD // 16

@functools.lru_cache(maxsize=None)
def _sc_mesh():
    return plsc.VectorSubcoreMesh(core_axis_name="c", subcore_axis_name="s")


DEGW = 128


NIB = DEG_PER_TILE // B


@functools.lru_cache(maxsize=None)
def _make_deg():
    @functools.partial(
        pl.kernel,
        out_type=jax.ShapeDtypeStruct((2, NPAD, DEGW), jnp.float32),
        mesh=_sc_mesh(),
        scratch_types=[
            pltpu.VMEM((NB, B), jnp.int32),
            pltpu.VMEM((NIB, B), jnp.int32),
            pltpu.VMEM((B, DEGW), jnp.float32),
            pltpu.VMEM((B, DEGW), jnp.float32),
            pltpu.VMEM_SHARED((NPAD, DEGW), jnp.float32),
            pltpu.SemaphoreType.DMA,
        ],
    )
    def _deg_kernel(dst_hbm, iota_hbm, ones_hbm, zeros_hbm, out_hbm,
                    didx, iidx, ones_v, buf, acc, sem):
        cid = lax.axis_index("c")
        sid = lax.axis_index("s")
        wid = sid * 2 + cid
        row0 = sid * DEG_PER_TILE
        pltpu.sync_copy(zeros_hbm, buf)
        pltpu.sync_copy(ones_hbm, ones_v)
        pltpu.sync_copy(dst_hbm.at[wid], didx)
        pltpu.sync_copy(iota_hbm.at[pl.ds(sid * NIB, NIB)], iidx)
        for j in range(NIB):
            pltpu.sync_copy(buf, acc.at[iidx.at[j]])
        plsc.subcore_barrier()

        @pl.loop(0, NB)
        def _edge_batch(k):
            pltpu.sync_copy(ones_v, acc.at[didx.at[k]], add=True)

        plsc.subcore_barrier()
        for j in range(NIB):
            pltpu.async_copy(acc.at[iidx.at[j]], buf, sem).wait()
            pltpu.sync_copy(
                buf, out_hbm.at[cid].at[pl.ds(row0 + j * B, B)])

    return _deg_kernel


@functools.lru_cache(maxsize=None)
def _make_scatter(nchunks):
    @functools.partial(
        pl.kernel,
        out_type=jax.ShapeDtypeStruct((2, nchunks, NPAD, C), jnp.float32),
        mesh=_sc_mesh(),
        scratch_types=[
            pltpu.VMEM((NB, B), jnp.int32),
            pltpu.VMEM((NB, B), jnp.int32),
            pltpu.VMEM((SNIB, B), jnp.int32),
            pltpu.VMEM((B, C), jnp.float32),
            pltpu.VMEM_SHARED((NPAD, C), jnp.float32),
            pltpu.SemaphoreType.DMA,
        ],
    )
    def _scatter_kernel(g_hbm, src_hbm, dst_hbm, zeros_hbm, iota_hbm, out_hbm,
                        sidx, didx, iidx, rows, acc, sem):
        cid = lax.axis_index("c")
        sid = lax.axis_index("s")
        wid = sid * 2 + cid
        row0 = sid * ROWS_PER_TILE
        pltpu.sync_copy(src_hbm.at[wid], sidx)
        pltpu.sync_copy(dst_hbm.at[wid], didx)
        pltpu.sync_copy(iota_hbm.at[pl.ds(sid * SNIB, SNIB)], iidx)
        pltpu.sync_copy(zeros_hbm, rows)
        for j in range(SNIB):
            pltpu.sync_copy(rows, acc.at[iidx.at[j]])
        plsc.subcore_barrier()

        for cc in range(nchunks):
            @pl.loop(0, NB)
            def _edge_batch(k):
                pltpu.async_copy(g_hbm.at[cc].at[sidx.at[k]], rows, sem).wait()
                pltpu.sync_copy(rows, acc.at[didx.at[k]], add=True)

            plsc.subcore_barrier()
            for j in range(SNIB):
                pltpu.async_copy(acc.at[iidx.at[j]], rows, sem).wait()
                pltpu.sync_copy(
                    rows, out_hbm.at[cid, cc].at[pl.ds(row0 + j * B, B)])
            if cc + 1 < nchunks:
                pltpu.sync_copy(zeros_hbm, rows)
                for j in range(SNIB):
                    pltpu.sync_copy(rows, acc.at[iidx.at[j]])
            plsc.subcore_barrier()

    return _scatter_kernel


def _dis_body(p0_ref, p1_ref, out_ref):
    deg = p0_ref[...] + p1_ref[...] + 1.0
    out_ref[...] = lax.rsqrt(deg)


def _dis(p0, p1):
    return pl.pallas_call(
        _dis_body,
        out_shape=jax.ShapeDtypeStruct((NPAD, 1), jnp.float32),
    )(p0, p1)


RB = 400
NRB = N // RB


def _matmul_body(x_ref, w_ref, dis_ref, out_ref):
    acc = jnp.dot(x_ref[...], w_ref[...], preferred_element_type=jnp.float32)
    out_ref[0] = acc * dis_ref[...]


def _matmul_scaled(x, w, dis):
    di, do = w.shape
    nchunks = do // C
    return pl.pallas_call(
        _matmul_body,
        grid=(nchunks, NRB),
        in_specs=[
            pl.BlockSpec((RB, di), lambda c, r: (r, 0)),
            pl.BlockSpec((di, C), lambda c, r: (0, c)),
            pl.BlockSpec((RB, 1), lambda c, r: (r, 0)),
        ],
        out_specs=pl.BlockSpec((1, RB, C), lambda c, r: (c, r, 0)),
        out_shape=jax.ShapeDtypeStruct((nchunks, N, C), jnp.float32),
    )(x, w, dis)


def _combine_body(g_ref, s0_ref, s1_ref, dis_ref, b_ref, y_ref, st_ref):
    r = pl.program_id(1)
    y = dis_ref[...] * (g_ref[0] + s0_ref[0] + s1_ref[0]) + b_ref[...]
    y_ref[...] = y

    @pl.when(r == 0)
    def _init():
        st_ref[...] = jnp.zeros_like(st_ref)

    st_ref[0:1, :] += jnp.sum(y, axis=0, keepdims=True)
    st_ref[1:2, :] += jnp.sum(y * y, axis=0, keepdims=True)


def _combine(g, s0, s1, dis, b):
    nchunks = g.shape[0]
    do = nchunks * C
    return pl.pallas_call(
        _combine_body,
        grid=(nchunks, NRB),
        in_specs=[
            pl.BlockSpec((1, RB, C), lambda c, r: (c, r, 0)),
            pl.BlockSpec((1, RB, C), lambda c, r: (c, r, 0)),
            pl.BlockSpec((1, RB, C), lambda c, r: (c, r, 0)),
            pl.BlockSpec((RB, 1), lambda c, r: (r, 0)),
            pl.BlockSpec((1, C), lambda c, r: (0, c)),
        ],
        out_specs=[
            pl.BlockSpec((RB, C), lambda c, r: (r, c)),
            pl.BlockSpec((2, C), lambda c, r: (0, c)),
        ],
        out_shape=[
            jax.ShapeDtypeStruct((N, do), jnp.float32),
            jax.ShapeDtypeStruct((2, do), jnp.float32),
        ],
    )(g, s0, s1, dis, b)


def _norm_body(y_ref, st_ref, gw_ref, gb_ref, gm_ref, out_ref, *, act):
    inv_n = 1.0 / N
    mean = st_ref[0:1, :] * inv_n
    ey2 = st_ref[1:2, :] * inv_n
    gm = gm_ref[...]
    var = ey2 - (2.0 * gm - gm * gm) * mean * mean
    scale = lax.rsqrt(var + 1e-5) * gw_ref[...]
    z = (y_ref[...] - gm * mean) * scale + gb_ref[...]
    out_ref[...] = jnp.maximum(z, 0.0) if act == "relu" else jnp.tanh(z)


def _graph_norm(y, st, gw, gb, gm, act):
    do = y.shape[1]
    nchunks = do // C
    return pl.pallas_call(
        functools.partial(_norm_body, act=act),
        grid=(nchunks, NRB),
        in_specs=[
            pl.BlockSpec((RB, C), lambda c, r: (r, c)),
            pl.BlockSpec((2, C), lambda c, r: (0, c)),
            pl.BlockSpec((1, C), lambda c, r: (0, c)),
            pl.BlockSpec((1, C), lambda c, r: (0, c)),
            pl.BlockSpec((1, C), lambda c, r: (0, c)),
        ],
        out_specs=pl.BlockSpec((RB, C), lambda c, r: (r, c)),
        out_shape=jax.ShapeDtypeStruct((N, do), jnp.float32),
    )(y, st, gw, gb, gm)


def _layer(x, src3, dst3, zeros_chunk, iota_rows, dis, w, b, gw, gb, gm, act):
    g = _matmul_scaled(x, w, dis)
    nchunks = g.shape[0]
    s = _make_scatter(nchunks)(g, src3, dst3, zeros_chunk, iota_rows)
    y, st = _combine(g, s[0], s[1], dis, b.reshape(1, -1))
    return _graph_norm(y, st, gw.reshape(1, -1), gb.reshape(1, -1),
                       gm.reshape(1, -1), act)


def kernel(x, edge_index, W1, b1, gw1, gb1, gm1, W2, b2, gw2, gb2, gm2,
           W3, b3, gw3, gb3, gm3):
    src3 = edge_index[0].astype(jnp.int32).reshape(NW, NB, B)
    dst3 = edge_index[1].astype(jnp.int32).reshape(NW, NB, B)

    ones_b = jnp.ones((B, DEGW), jnp.float32)
    zeros_deg = jnp.zeros((B, DEGW), jnp.float32)
    zeros_chunk = jnp.zeros((B, C), jnp.float32)
    iota_rows = jnp.arange(NPAD, dtype=jnp.int32).reshape(16 * NIB, B)

    degp = _make_deg()(dst3, iota_rows, ones_b, zeros_deg)
    dis_pad = _dis(degp[0, :, :1], degp[1, :, :1])
    dis = dis_pad[:N]

    h = _layer(x, src3, dst3, zeros_chunk, iota_rows, dis,
               W1, b1, gw1, gb1, gm1, "relu")
    h = _layer(h, src3, dst3, zeros_chunk, iota_rows, dis,
               W2, b2, gw2, gb2, gm2, "relu")
    h = _layer(h, src3, dst3, zeros_chunk, iota_rows, dis,
               W3, b3, gw3, gb3, gm3, "tanh")
    return h


# speedup vs baseline: 12.0055x; 1.4199x over previous
import functools

import jax
import jax.numpy as jnp
from jax import lax
from jax.experimental import pallas as pl
from jax.experimental.pallas import tpu as pltpu
from jax.experimental.pallas import tpu_sc as plsc

N = 10000
E = 320000
NPAD = 10240
C = 128
NW = 32
EPW = E // NW
B = 80
NB = EPW // B
ROWS_PER_TILE = NPAD // 16
SNIB = ROWS_PER_TILE // B
HNB = 64
DEG_PER_TILE = NPAD // 16

@functools.lru_cache(maxsize=None)
def _sc_mesh():
    return plsc.VectorSubcoreMesh(core_axis_name="c", subcore_axis_name="s")


DEGW = 128


NIB = DEG_PER_TILE // B


@functools.lru_cache(maxsize=None)
def _make_deg():
    @functools.partial(
        pl.kernel,
        out_type=jax.ShapeDtypeStruct((2, NPAD, DEGW), jnp.float32),
        mesh=_sc_mesh(),
        scratch_types=[
            pltpu.VMEM((NB, B), jnp.int32),
            pltpu.VMEM((NIB, B), jnp.int32),
            pltpu.VMEM((B, DEGW), jnp.float32),
            pltpu.VMEM((B, DEGW), jnp.float32),
            pltpu.VMEM_SHARED((NPAD, DEGW), jnp.float32),
            pltpu.SemaphoreType.DMA,
        ],
    )
    def _deg_kernel(dst_hbm, iota_hbm, ones_hbm, zeros_hbm, out_hbm,
                    didx, iidx, ones_v, buf, acc, sem):
        cid = lax.axis_index("c")
        sid = lax.axis_index("s")
        wid = sid * 2 + cid
        row0 = sid * DEG_PER_TILE
        pltpu.sync_copy(zeros_hbm, buf)
        pltpu.sync_copy(ones_hbm, ones_v)
        pltpu.sync_copy(dst_hbm.at[wid], didx)
        pltpu.sync_copy(iota_hbm.at[pl.ds(sid * NIB, NIB)], iidx)
        for j in range(NIB):
            pltpu.sync_copy(buf, acc.at[iidx.at[j]])
        plsc.subcore_barrier()

        @pl.loop(0, NB)
        def _edge_batch(k):
            pltpu.sync_copy(ones_v, acc.at[didx.at[k]], add=True)

        plsc.subcore_barrier()
        for j in range(NIB):
            pltpu.async_copy(acc.at[iidx.at[j]], buf, sem).wait()
            pltpu.sync_copy(
                buf, out_hbm.at[cid].at[pl.ds(row0 + j * B, B)])

    return _deg_kernel


@functools.lru_cache(maxsize=None)
def _make_scatter(nchunks):
    @functools.partial(
        pl.kernel,
        out_type=jax.ShapeDtypeStruct((2, nchunks, NPAD, C), jnp.float32),
        mesh=_sc_mesh(),
        scratch_types=[
            pltpu.VMEM((HNB, B), jnp.int32),
            pltpu.VMEM((HNB, B), jnp.int32),
            pltpu.VMEM((SNIB, B), jnp.int32),
            pltpu.VMEM((B, C), jnp.float32),
            pltpu.VMEM((B, C), jnp.float32),
            pltpu.VMEM_SHARED((NPAD, C), jnp.float32),
            pltpu.SemaphoreType.DMA,
            pltpu.SemaphoreType.DMA,
        ],
    )
    def _scatter_kernel(g_hbm, src_hbm, dst_hbm, zeros_hbm, iota_hbm, out_hbm,
                        sidx, didx, iidx, rows_a, rows_b, acc, sem_a, sem_b):
        cid = lax.axis_index("c")
        sid = lax.axis_index("s")
        wid = sid * 2 + cid
        row0 = sid * ROWS_PER_TILE
        pltpu.sync_copy(iota_hbm.at[pl.ds(sid * SNIB, SNIB)], iidx)
        pltpu.sync_copy(zeros_hbm, rows_a)
        for j in range(SNIB):
            pltpu.sync_copy(rows_a, acc.at[iidx.at[j]])
        plsc.subcore_barrier()

        for cc in range(nchunks):
            for h, nb in ((0, HNB), (1, NB - HNB)):
                base = h * HNB
                pltpu.sync_copy(src_hbm.at[wid, pl.ds(base, nb)],
                                sidx.at[pl.ds(0, nb)])
                pltpu.sync_copy(dst_hbm.at[wid, pl.ds(base, nb)],
                                didx.at[pl.ds(0, nb)])

                def gat(k, buf, sem):
                    pltpu.async_copy(g_hbm.at[cc].at[sidx.at[k]], buf, sem)

                def wait(buf, sem):
                    pltpu.make_async_copy(
                        g_hbm.at[cc].at[sidx.at[0]], buf, sem).wait()

                gat(0, rows_a, sem_a)

                @pl.loop(0, nb // 2)
                def _pair(k2):
                    k = 2 * k2
                    gat(k + 1, rows_b, sem_b)
                    wait(rows_a, sem_a)
                    pltpu.sync_copy(rows_a, acc.at[didx.at[k]], add=True)

                    @pl.when(k + 2 < nb)
                    def _more():
                        gat(k + 2, rows_a, sem_a)

                    wait(rows_b, sem_b)
                    pltpu.sync_copy(rows_b, acc.at[didx.at[k + 1]], add=True)

                if nb % 2:
                    wait(rows_a, sem_a)
                    pltpu.sync_copy(rows_a, acc.at[didx.at[nb - 1]], add=True)

            plsc.subcore_barrier()
            for j in range(SNIB):
                pltpu.async_copy(acc.at[iidx.at[j]], rows_a, sem_a).wait()
                pltpu.sync_copy(
                    rows_a, out_hbm.at[cid, cc].at[pl.ds(row0 + j * B, B)])
            if cc + 1 < nchunks:
                pltpu.sync_copy(zeros_hbm, rows_a)
                for j in range(SNIB):
                    pltpu.sync_copy(rows_a, acc.at[iidx.at[j]])
            plsc.subcore_barrier()

    return _scatter_kernel


def _dis_body(p0_ref, p1_ref, out_ref):
    deg = p0_ref[...] + p1_ref[...] + 1.0
    out_ref[...] = lax.rsqrt(deg)


def _dis(p0, p1):
    return pl.pallas_call(
        _dis_body,
        out_shape=jax.ShapeDtypeStruct((NPAD, 1), jnp.float32),
    )(p0, p1)


RB = 400
NRB = N // RB


def _matmul_body(x_ref, w_ref, dis_ref, out_ref):
    acc = jnp.dot(x_ref[...], w_ref[...], preferred_element_type=jnp.float32)
    out_ref[0] = acc * dis_ref[...]


def _matmul_scaled(x, w, dis):
    di, do = w.shape
    nchunks = do // C
    return pl.pallas_call(
        _matmul_body,
        grid=(nchunks, NRB),
        in_specs=[
            pl.BlockSpec((RB, di), lambda c, r: (r, 0)),
            pl.BlockSpec((di, C), lambda c, r: (0, c)),
            pl.BlockSpec((RB, 1), lambda c, r: (r, 0)),
        ],
        out_specs=pl.BlockSpec((1, RB, C), lambda c, r: (c, r, 0)),
        out_shape=jax.ShapeDtypeStruct((nchunks, N, C), jnp.float32),
    )(x, w, dis)


def _combine_body(g_ref, s0_ref, s1_ref, dis_ref, b_ref, y_ref, st_ref):
    r = pl.program_id(1)
    y = dis_ref[...] * (g_ref[0] + s0_ref[0] + s1_ref[0]) + b_ref[...]
    y_ref[...] = y

    @pl.when(r == 0)
    def _init():
        st_ref[...] = jnp.zeros_like(st_ref)

    st_ref[0:1, :] += jnp.sum(y, axis=0, keepdims=True)
    st_ref[1:2, :] += jnp.sum(y * y, axis=0, keepdims=True)


def _combine(g, s0, s1, dis, b):
    nchunks = g.shape[0]
    do = nchunks * C
    return pl.pallas_call(
        _combine_body,
        grid=(nchunks, NRB),
        in_specs=[
            pl.BlockSpec((1, RB, C), lambda c, r: (c, r, 0)),
            pl.BlockSpec((1, RB, C), lambda c, r: (c, r, 0)),
            pl.BlockSpec((1, RB, C), lambda c, r: (c, r, 0)),
            pl.BlockSpec((RB, 1), lambda c, r: (r, 0)),
            pl.BlockSpec((1, C), lambda c, r: (0, c)),
        ],
        out_specs=[
            pl.BlockSpec((RB, C), lambda c, r: (r, c)),
            pl.BlockSpec((2, C), lambda c, r: (0, c)),
        ],
        out_shape=[
            jax.ShapeDtypeStruct((N, do), jnp.float32),
            jax.ShapeDtypeStruct((2, do), jnp.float32),
        ],
    )(g, s0, s1, dis, b)


def _norm_body(y_ref, st_ref, gw_ref, gb_ref, gm_ref, out_ref, *, act):
    inv_n = 1.0 / N
    mean = st_ref[0:1, :] * inv_n
    ey2 = st_ref[1:2, :] * inv_n
    gm = gm_ref[...]
    var = ey2 - (2.0 * gm - gm * gm) * mean * mean
    scale = lax.rsqrt(var + 1e-5) * gw_ref[...]
    z = (y_ref[...] - gm * mean) * scale + gb_ref[...]
    out_ref[...] = jnp.maximum(z, 0.0) if act == "relu" else jnp.tanh(z)


def _graph_norm(y, st, gw, gb, gm, act):
    do = y.shape[1]
    nchunks = do // C
    return pl.pallas_call(
        functools.partial(_norm_body, act=act),
        grid=(nchunks, NRB),
        in_specs=[
            pl.BlockSpec((RB, C), lambda c, r: (r, c)),
            pl.BlockSpec((2, C), lambda c, r: (0, c)),
            pl.BlockSpec((1, C), lambda c, r: (0, c)),
            pl.BlockSpec((1, C), lambda c, r: (0, c)),
            pl.BlockSpec((1, C), lambda c, r: (0, c)),
        ],
        out_specs=pl.BlockSpec((RB, C), lambda c, r: (r, c)),
        out_shape=jax.ShapeDtypeStruct((N, do), jnp.float32),
    )(y, st, gw, gb, gm)


def _layer(x, src3, dst3, zeros_chunk, iota_rows, dis, w, b, gw, gb, gm, act):
    g = _matmul_scaled(x, w, dis)
    nchunks = g.shape[0]
    s = _make_scatter(nchunks)(g, src3, dst3, zeros_chunk, iota_rows)
    y, st = _combine(g, s[0], s[1], dis, b.reshape(1, -1))
    return _graph_norm(y, st, gw.reshape(1, -1), gb.reshape(1, -1),
                       gm.reshape(1, -1), act)


def kernel(x, edge_index, W1, b1, gw1, gb1, gm1, W2, b2, gw2, gb2, gm2,
           W3, b3, gw3, gb3, gm3):
    src3 = edge_index[0].astype(jnp.int32).reshape(NW, NB, B)
    dst3 = edge_index[1].astype(jnp.int32).reshape(NW, NB, B)

    ones_b = jnp.ones((B, DEGW), jnp.float32)
    zeros_deg = jnp.zeros((B, DEGW), jnp.float32)
    zeros_chunk = jnp.zeros((B, C), jnp.float32)
    iota_rows = jnp.arange(NPAD, dtype=jnp.int32).reshape(16 * NIB, B)

    degp = _make_deg()(dst3, iota_rows, ones_b, zeros_deg)
    dis_pad = _dis(degp[0, :, :1], degp[1, :, :1])
    dis = dis_pad[:N]

    h = _layer(x, src3, dst3, zeros_chunk, iota_rows, dis,
               W1, b1, gw1, gb1, gm1, "relu")
    h = _layer(h, src3, dst3, zeros_chunk, iota_rows, dis,
               W2, b2, gw2, gb2, gm2, "relu")
    h = _layer(h, src3, dst3, zeros_chunk, iota_rows, dis,
               W3, b3, gw3, gb3, gm3, "tanh")
    return h
